# baseline (speedup 1.0000x reference)
import functools

import jax
import jax.numpy as jnp
from jax import lax
from jax.experimental import pallas as pl
from jax.experimental.pallas import tpu as pltpu
from jax.experimental.pallas import tpu_sc as plsc


_QBLK = 2048
_QTOT = 1 << 18


def _rowdot_body(we_ref, wm_ref, out_ref):
    p = we_ref[...] * wm_ref[...]
    a = jax.lax.broadcasted_iota(jnp.int32, (4, 128), 0)
    u = jax.lax.broadcasted_iota(jnp.int32, (4, 128), 1)
    e4 = jnp.where(u // 32 == a, 1.0, 0.0).astype(jnp.float32)
    out_ref[...] = jax.lax.dot_general(
        e4, p, (((1,), (1,)), ((), ())),
        preferred_element_type=jnp.float32)


def _rowdot(we, wm):
    n, d = we.shape
    rows = n * d // 128
    grid = pl.cdiv(rows, _QBLK)
    out2d = pl.pallas_call(
        _rowdot_body,
        grid=(grid,),
        in_specs=[
            pl.BlockSpec((_QBLK, 128), lambda i: (i, 0)),
            pl.BlockSpec((_QBLK, 128), lambda i: (i, 0)),
        ],
        out_specs=pl.BlockSpec((4, _QBLK), lambda i: (0, i)),
        out_shape=jax.ShapeDtypeStruct((4, _QTOT), jnp.float32),
    )(we.reshape(rows, 128), wm.reshape(rows, 128))
    return out2d.reshape(-1)


_CHUNK = 12800


@functools.cache
def _make_gather(n_idx):
    info = plsc.get_sparse_core_info()
    nc, ns = info.num_cores, info.num_subcores
    nw = nc * ns
    per_w = n_idx // nw
    n_ch = per_w // _CHUNK
    mesh = plsc.VectorSubcoreMesh(core_axis_name="c", subcore_axis_name="s")

    @functools.partial(
        pl.kernel,
        mesh=mesh,
        out_type=jax.ShapeDtypeStruct((n_idx,), jnp.float32),
        scratch_types=[
            pltpu.VMEM((_CHUNK,), jnp.int32),
            pltpu.VMEM((_CHUNK,), jnp.float32),
            pltpu.SemaphoreType.DMA,
        ],
    )
    def gather_k(rowdot_hbm, xf_hbm, out_hbm, idx_v, val_v, sem):
        wid = lax.axis_index("s") * nc + lax.axis_index("c")
        base = wid * per_w

        def body(i, carry):
            off = base + i * _CHUNK
            pltpu.sync_copy(xf_hbm.at[pl.ds(off, _CHUNK)], idx_v)

            def tbody(j, c):
                t = idx_v[pl.ds(j * 16, 16)]
                idx_v[pl.ds(j * 16, 16)] = ((t & 3) << 18) | (t >> 2)
                return c

            lax.fori_loop(0, _CHUNK // 16, tbody, 0, unroll=8)
            pltpu.async_copy(rowdot_hbm.at[idx_v], val_v, sem).wait()
            pltpu.sync_copy(val_v, out_hbm.at[pl.ds(off, _CHUNK)])
            return carry

        lax.fori_loop(0, n_ch, body, 0)

    return gather_k


def kernel(x, W_embed, W_masks):
    rowdot = _rowdot(W_embed, W_masks)
    xf = x.reshape(-1)
    out = _make_gather(xf.shape[0])(rowdot, xf)
    return out.reshape(x.shape)


# speedup vs baseline: 4.7477x; 4.7477x over previous
import functools

import jax
import jax.numpy as jnp
from jax import lax
from jax.experimental import pallas as pl
from jax.experimental.pallas import tpu as pltpu
from jax.experimental.pallas import tpu_sc as plsc


_VBLK = 1 << 15


def _rowdot_body(weT_ref, wmT_ref, out_ref):
    out_ref[...] = jnp.sum(weT_ref[...] * wmT_ref[...], axis=0)


def _rowdot(we, wm):
    n, d = we.shape
    grid = pl.cdiv(n, _VBLK)
    return pl.pallas_call(
        _rowdot_body,
        grid=(grid,),
        in_specs=[
            pl.BlockSpec((d, _VBLK), lambda i: (0, i)),
            pl.BlockSpec((d, _VBLK), lambda i: (0, i)),
        ],
        out_specs=pl.BlockSpec((_VBLK,), lambda i: (i,)),
        out_shape=jax.ShapeDtypeStruct((grid * _VBLK,), jnp.float32),
    )(we.T, wm.T)


_CHUNK = 12800


@functools.cache
def _make_gather(n_idx):
    info = plsc.get_sparse_core_info()
    nc, ns = info.num_cores, info.num_subcores
    nw = nc * ns
    per_w = n_idx // nw
    n_ch = per_w // _CHUNK
    mesh = plsc.VectorSubcoreMesh(core_axis_name="c", subcore_axis_name="s")

    @functools.partial(
        pl.kernel,
        mesh=mesh,
        out_type=jax.ShapeDtypeStruct((n_idx,), jnp.float32),
        scratch_types=[
            pltpu.VMEM((_CHUNK,), jnp.int32),
            pltpu.VMEM((_CHUNK,), jnp.float32),
            pltpu.SemaphoreType.DMA,
        ],
    )
    def gather_k(rowdot_hbm, xf_hbm, out_hbm, idx_v, val_v, sem):
        wid = lax.axis_index("s") * nc + lax.axis_index("c")
        base = wid * per_w

        def body(i, carry):
            off = base + i * _CHUNK
            pltpu.sync_copy(xf_hbm.at[pl.ds(off, _CHUNK)], idx_v)
            pltpu.async_copy(rowdot_hbm.at[idx_v], val_v, sem).wait()
            pltpu.sync_copy(val_v, out_hbm.at[pl.ds(off, _CHUNK)])
            return carry

        lax.fori_loop(0, n_ch, body, 0)

    return gather_k


def kernel(x, W_embed, W_masks):
    rowdot = _rowdot(W_embed, W_masks)
    xf = x.T.reshape(-1)
    out = _make_gather(xf.shape[0])(rowdot, xf)
    return out.reshape(x.shape[1], x.shape[0]).T
